# baseline (speedup 1.0000x reference)
import functools

import jax
import jax.numpy as jnp
from jax import lax
from jax.experimental import pallas as pl
from jax.experimental.pallas import tpu as pltpu
from jax.experimental.pallas import tpu_sc as plsc

N_VOCAB = 1000000
D_EMB = 32
N_I = 4096
N_J = 200

_info = plsc.get_sparse_core_info()
NC = _info.num_cores
NS = _info.num_subcores
NW = NC * NS

LANE = 16

N_TCOL = 7813
TCOL_MAIN = N_TCOL // NW
SCRATCH_ROWS = N_TCOL * 128
A_NBUF = 4
A_NRB = 2


def _iota16():
    return lax.iota(jnp.int32, LANE)


def _reformat_kernel(tt_hbm, scr_hbm, in0, in1, in2, in3, rb0, rb1,
                     g0, g1, g2, g3, s0, s1):
    wid = lax.axis_index("s") * NC + lax.axis_index("c")
    in_v = [in0, in1, in2, in3]
    rb_v = [rb0, rb1]
    gsems = [g0, g1, g2, g3]
    ssems = [s0, s1]

    def fire_in(c, b):
        pltpu.async_copy(
            tt_hbm.at[:, pl.ds(c * 128, 128)], in_v[b], gsems[b]
        )

    def wait_in(c, b):
        pltpu.make_async_copy(
            tt_hbm.at[:, pl.ds(c * 128, 128)], in_v[b], gsems[b]
        ).wait()

    def fire_out(c, rb):
        pltpu.async_copy(
            rb_v[rb], scr_hbm.at[pl.ds(c * 4096, 4096)], ssems[rb]
        )

    def wait_out(c, rb):
        pltpu.make_async_copy(
            rb_v[rb], scr_hbm.at[pl.ds(c * 4096, 4096)], ssems[rb]
        ).wait()

    iv32 = _iota16() * 32

    def transpose(b, rb):
        for d in range(32):
            for l0 in range(8):
                x = in_v[b][d, pl.ds(l0 * 16, 16)]
                plsc.store_scatter(rb_v[rb], [iv32 + (l0 * 512 + d)], x)

    def col(k):
        return wid + k * NW

    LA = 2
    for k in range(LA):
        fire_in(col(k), k % A_NBUF)

    @pl.loop(0, TCOL_MAIN, step=A_NBUF)
    def _main(kb):
        for b in range(A_NBUF):
            k = kb + b
            wait_in(col(k), b)

            @pl.when(k + LA < TCOL_MAIN)
            def _():
                fire_in(col(k + LA), (b + LA) % A_NBUF)

            rb = b % A_NRB

            @pl.when(k >= A_NRB)
            def _():
                wait_out(col(k - A_NRB), rb)

            transpose(b, rb)
            fire_out(col(k), rb)

    for k in range(TCOL_MAIN - A_NRB, TCOL_MAIN):
        wait_out(col(k), (k % A_NBUF) % A_NRB)

    @pl.when(wid < N_TCOL - TCOL_MAIN * NW)
    def _rem():
        k = TCOL_MAIN
        b = k % A_NBUF
        rb = b % A_NRB
        fire_in(col(k), b)
        wait_in(col(k), b)
        transpose(b, rb)
        fire_out(col(k), rb)
        wait_out(col(k), rb)


@jax.jit
def _reformat(tt):
    mesh = plsc.VectorSubcoreMesh(core_axis_name="c", subcore_axis_name="s")
    run = functools.partial(
        pl.kernel,
        out_type=jax.ShapeDtypeStruct((SCRATCH_ROWS * D_EMB,), jnp.float32),
        mesh=mesh,
        scratch_types=(
            [pltpu.VMEM((32, 128), jnp.float32)] * A_NBUF
            + [pltpu.VMEM((4096,), jnp.float32)] * A_NRB
            + [pltpu.SemaphoreType.DMA] * (A_NBUF + A_NRB)
        ),
        compiler_params=pltpu.CompilerParams(
            use_tc_tiling_on_sc=True, needs_layout_passes=False
        ),
    )(_reformat_kernel)
    return run(tt)


R_PER_W = N_I // NW
B_NBUF = 6
B_NT = 2
B_LA = 4


def _gather_kernel(scr_hbm, x4_hbm, o5_hbm, idx_v, rows_v, t_v, gsems, ssems):
    wid = lax.axis_index("s") * NC + lax.axis_index("c")

    pltpu.sync_copy(x4_hbm.at[:, wid], idx_v)

    def fire_g(j, b):
        jt, js = j // 8, j % 8
        pltpu.async_copy(
            scr_hbm.at[idx_v.at[jt, js]], rows_v.at[b], gsems.at[b]
        )

    def wait_g(j, b):
        jt, js = j // 8, j % 8
        pltpu.make_async_copy(
            scr_hbm.at[idx_v.at[jt, js]], rows_v.at[b], gsems.at[b]
        ).wait()

    def fire_s(j, tb):
        for dt in range(4):
            pltpu.async_copy(
                t_v.at[tb, dt], o5_hbm.at[j, dt, wid], ssems.at[tb]
            )

    def wait_s(j, tb):
        for dt in range(4):
            pltpu.make_async_copy(
                t_v.at[tb, dt], o5_hbm.at[j, dt, wid], ssems.at[tb]
            ).wait()

    iv32 = _iota16() * 32

    def transpose(b, tb):
        for d in range(32):
            for l0 in range(8):
                x = plsc.load_gather(
                    rows_v.at[b], [_iota16() + l0 * 16, jnp.full((16,), d, jnp.int32)]
                )
                t_v[tb, d // 8, d % 8, pl.ds(l0 * 16, 16)] = x

    for j in range(B_LA):
        fire_g(j, j % B_NBUF)

    @pl.loop(0, N_J - 2, step=B_NBUF)
    def _main(jb):
        for b in range(B_NBUF):
            j = jb + b
            wait_g(j, b)

            @pl.when(j + B_LA < N_J)
            def _():
                fire_g(j + B_LA, (b + B_LA) % B_NBUF)

            tb = b % B_NT

            @pl.when(j >= B_NT)
            def _():
                wait_s(j - B_NT, tb)

            transpose(b, tb)
            fire_s(j, tb)

    for j in range(N_J - 2, N_J):
        b = j % B_NBUF
        tb = b % B_NT
        wait_g(j, b)
        wait_s(j - B_NT, tb)
        transpose(b, tb)
        fire_s(j, tb)

    for j in range(N_J - B_NT, N_J):
        wait_s(j, (j % B_NBUF) % B_NT)


@jax.jit
def _emb(x, table):
    tt = table.T
    scratch = _reformat(tt)
    x4 = x.astype(jnp.int32).T.reshape(25, 8, 32, 128).transpose(0, 2, 1, 3)
    mesh = plsc.VectorSubcoreMesh(core_axis_name="c", subcore_axis_name="s")
    run = functools.partial(
        pl.kernel,
        out_type=jax.ShapeDtypeStruct((N_J, 4, NW, 8, 128), jnp.float32),
        mesh=mesh,
        scratch_types=[
            pltpu.VMEM((25, 8, 128), jnp.int32),
            pltpu.VMEM((B_NBUF, 128, 32), jnp.float32),
            pltpu.VMEM((B_NT, 4, 8, 128), jnp.float32),
            pltpu.SemaphoreType.DMA((B_NBUF,)),
            pltpu.SemaphoreType.DMA((B_NT,)),
        ],
        compiler_params=pltpu.CompilerParams(
            use_tc_tiling_on_sc=False, needs_layout_passes=False
        ),
    )(_gather_kernel)
    o5 = run(scratch.reshape(SCRATCH_ROWS, D_EMB), x4)
    return o5.transpose(2, 4, 0, 1, 3).reshape(N_I, N_J, D_EMB)


def kernel(x, table):
    return _emb(x, table)


# speedup vs baseline: 1.6844x; 1.6844x over previous
import functools

import jax
import jax.numpy as jnp
from jax import lax
from jax.experimental import pallas as pl
from jax.experimental.pallas import tpu as pltpu
from jax.experimental.pallas import tpu_sc as plsc

N_VOCAB = 1000000
D_EMB = 32
N_I = 4096
N_J = 200

_info = plsc.get_sparse_core_info()
NC = _info.num_cores
NS = _info.num_subcores
NW = NC * NS

LANE = 16

N_TCOL = 7813
TCOL_MAIN = N_TCOL // NW
SCRATCH_ROWS = N_TCOL * 128
A_NBUF = 4
A_NRB = 2


def _iota16():
    return lax.iota(jnp.int32, LANE)


def _reformat_kernel(tt_hbm, scr_hbm, in0, in1, in2, in3, rb0, rb1,
                     g0, g1, g2, g3, s0, s1):
    wid = lax.axis_index("s") * NC + lax.axis_index("c")
    in_v = [in0, in1, in2, in3]
    rb_v = [rb0, rb1]
    gsems = [g0, g1, g2, g3]
    ssems = [s0, s1]

    def fire_in(c, b):
        pltpu.async_copy(
            tt_hbm.at[:, pl.ds(c * 128, 128)], in_v[b], gsems[b]
        )

    def wait_in(c, b):
        pltpu.make_async_copy(
            tt_hbm.at[:, pl.ds(c * 128, 128)], in_v[b], gsems[b]
        ).wait()

    def fire_out(c, rb):
        pltpu.async_copy(
            rb_v[rb], scr_hbm.at[pl.ds(c * 4096, 4096)], ssems[rb]
        )

    def wait_out(c, rb):
        pltpu.make_async_copy(
            rb_v[rb], scr_hbm.at[pl.ds(c * 4096, 4096)], ssems[rb]
        ).wait()

    iv32 = _iota16() * 32
    ivs = [iv32 + l0 * 512 for l0 in range(8)]

    def transpose(b, rb):
        @plsc.parallel_loop(0, 32, unroll=4)
        def _t(d):
            for l0 in range(8):
                x = in_v[b][d, pl.ds(l0 * 16, 16)]
                plsc.store_scatter(rb_v[rb], [ivs[l0] + d], x)

    def col(k):
        return wid + k * NW

    LA = 2
    for k in range(LA):
        fire_in(col(k), k % A_NBUF)

    @pl.loop(0, TCOL_MAIN, step=A_NBUF)
    def _main(kb):
        for b in range(A_NBUF):
            k = kb + b
            wait_in(col(k), b)

            @pl.when(k + LA < TCOL_MAIN)
            def _():
                fire_in(col(k + LA), (b + LA) % A_NBUF)

            rb = b % A_NRB

            @pl.when(k >= A_NRB)
            def _():
                wait_out(col(k - A_NRB), rb)

            transpose(b, rb)
            fire_out(col(k), rb)

    for k in range(TCOL_MAIN - A_NRB, TCOL_MAIN):
        wait_out(col(k), (k % A_NBUF) % A_NRB)

    @pl.when(wid < N_TCOL - TCOL_MAIN * NW)
    def _rem():
        k = TCOL_MAIN
        b = k % A_NBUF
        rb = b % A_NRB
        fire_in(col(k), b)
        wait_in(col(k), b)
        transpose(b, rb)
        fire_out(col(k), rb)
        wait_out(col(k), rb)


@jax.jit
def _reformat(tt):
    mesh = plsc.VectorSubcoreMesh(core_axis_name="c", subcore_axis_name="s")
    run = functools.partial(
        pl.kernel,
        out_type=jax.ShapeDtypeStruct((SCRATCH_ROWS * D_EMB,), jnp.float32),
        mesh=mesh,
        scratch_types=(
            [pltpu.VMEM((32, 128), jnp.float32)] * A_NBUF
            + [pltpu.VMEM((4096,), jnp.float32)] * A_NRB
            + [pltpu.SemaphoreType.DMA] * (A_NBUF + A_NRB)
        ),
        compiler_params=pltpu.CompilerParams(
            use_tc_tiling_on_sc=True, needs_layout_passes=False
        ),
    )(_reformat_kernel)
    return run(tt)


R_PER_W = N_I // NW
B_NBUF = 6
B_NT = 2
B_LA = 4


def _gather_kernel(scr_hbm, x4_hbm, o5_hbm, idx_v, rows_v, t_v, gsems, ssems):
    wid = lax.axis_index("s") * NC + lax.axis_index("c")

    pltpu.sync_copy(x4_hbm.at[:, wid], idx_v)

    def fire_g(j, b):
        jt, js = j // 8, j % 8
        pltpu.async_copy(
            scr_hbm.at[idx_v.at[jt, js]], rows_v.at[b], gsems.at[b]
        )

    def wait_g(j, b):
        jt, js = j // 8, j % 8
        pltpu.make_async_copy(
            scr_hbm.at[idx_v.at[jt, js]], rows_v.at[b], gsems.at[b]
        ).wait()

    def fire_s(j, tb):
        for dt in range(4):
            pltpu.async_copy(
                t_v.at[tb, pl.ds(dt * 1024, 1024)],
                o5_hbm.at[j, dt, wid],
                ssems.at[tb],
            )

    def wait_s(j, tb):
        for dt in range(4):
            pltpu.make_async_copy(
                t_v.at[tb, pl.ds(dt * 1024, 1024)],
                o5_hbm.at[j, dt, wid],
                ssems.at[tb],
            ).wait()

    iv128 = _iota16() * 128

    def transpose(b, tb):
        @plsc.parallel_loop(0, 128, unroll=8)
        def _t(l):
            x0 = rows_v[b, l, pl.ds(0, 16)]
            plsc.store_scatter(t_v.at[tb], [iv128 + l], x0)
            x1 = rows_v[b, l, pl.ds(16, 16)]
            plsc.store_scatter(t_v.at[tb], [iv128 + (2048 + l)], x1)

    for j in range(B_LA):
        fire_g(j, j % B_NBUF)

    @pl.loop(0, N_J - 2, step=B_NBUF)
    def _main(jb):
        for b in range(B_NBUF):
            j = jb + b
            wait_g(j, b)

            @pl.when(j + B_LA < N_J)
            def _():
                fire_g(j + B_LA, (b + B_LA) % B_NBUF)

            tb = b % B_NT

            @pl.when(j >= B_NT)
            def _():
                wait_s(j - B_NT, tb)

            transpose(b, tb)
            fire_s(j, tb)

    for j in range(N_J - 2, N_J):
        b = j % B_NBUF
        tb = b % B_NT
        wait_g(j, b)
        wait_s(j - B_NT, tb)
        transpose(b, tb)
        fire_s(j, tb)

    for j in range(N_J - B_NT, N_J):
        wait_s(j, (j % B_NBUF) % B_NT)


@jax.jit
def _emb(x, table):
    tt = table.T
    scratch = _reformat(tt)
    x4 = x.astype(jnp.int32).T.reshape(25, 8, 32, 128).transpose(0, 2, 1, 3)
    mesh = plsc.VectorSubcoreMesh(core_axis_name="c", subcore_axis_name="s")
    run = functools.partial(
        pl.kernel,
        out_type=jax.ShapeDtypeStruct((N_J, 4, NW, 1024), jnp.float32),
        mesh=mesh,
        scratch_types=[
            pltpu.VMEM((25, 8, 128), jnp.int32),
            pltpu.VMEM((B_NBUF, 128, 32), jnp.float32),
            pltpu.VMEM((B_NT, 4096), jnp.float32),
            pltpu.SemaphoreType.DMA((B_NBUF,)),
            pltpu.SemaphoreType.DMA((B_NT,)),
        ],
        compiler_params=pltpu.CompilerParams(
            use_tc_tiling_on_sc=False, needs_layout_passes=False
        ),
    )(_gather_kernel)
    o5 = run(scratch.reshape(SCRATCH_ROWS, D_EMB), x4)
    o6 = o5.reshape(N_J, 4, NW, 8, 128)
    return o6.transpose(2, 4, 0, 1, 3).reshape(N_I, N_J, D_EMB)


def kernel(x, table):
    return _emb(x, table)


# speedup vs baseline: 4.9998x; 2.9683x over previous
import functools

import jax
import jax.numpy as jnp
from jax import lax
from jax.experimental import pallas as pl
from jax.experimental.pallas import tpu as pltpu
from jax.experimental.pallas import tpu_sc as plsc

N_VOCAB = 1000000
D_EMB = 32
N_I = 4096
N_J = 200

_info = plsc.get_sparse_core_info()
NC = _info.num_cores
NS = _info.num_subcores
NW = NC * NS

LANE = 16

N_TCOL = 7813
TCOL_MAIN = N_TCOL // NW
SCRATCH_ROWS = N_TCOL * 128
A_NBUF = 4
A_NRB = 2


def _iota16():
    return lax.iota(jnp.int32, LANE)


def _reformat_kernel(tt_hbm, scr_hbm, in0, in1, in2, in3, rb0, rb1,
                     g0, g1, g2, g3, s0, s1):
    wid = lax.axis_index("s") * NC + lax.axis_index("c")
    in_v = [in0, in1, in2, in3]
    rb_v = [rb0, rb1]
    gsems = [g0, g1, g2, g3]
    ssems = [s0, s1]

    def fire_in(c, b):
        pltpu.async_copy(
            tt_hbm.at[:, pl.ds(c * 128, 128)], in_v[b], gsems[b]
        )

    def wait_in(c, b):
        pltpu.make_async_copy(
            tt_hbm.at[:, pl.ds(c * 128, 128)], in_v[b], gsems[b]
        ).wait()

    def fire_out(c, rb):
        pltpu.async_copy(
            rb_v[rb], scr_hbm.at[pl.ds(c * 4096, 4096)], ssems[rb]
        )

    def wait_out(c, rb):
        pltpu.make_async_copy(
            rb_v[rb], scr_hbm.at[pl.ds(c * 4096, 4096)], ssems[rb]
        ).wait()

    l_idx = [_iota16() + l0 * 16 for l0 in range(8)]
    l32 = [(_iota16() + l0 * 16) * 32 for l0 in range(8)]

    def transpose(b, rb):
        @plsc.parallel_loop(0, 32, unroll=4)
        def _t(d0):
            dmod = (d0 + _iota16()) & 31
            for l0 in range(8):
                x = plsc.load_gather(in_v[b], [dmod, l_idx[l0]])
                plsc.store_scatter(rb_v[rb], [l32[l0] + dmod], x)

    def col(k):
        return wid + k * NW

    LA = 2
    for k in range(LA):
        fire_in(col(k), k % A_NBUF)

    @pl.loop(0, TCOL_MAIN, step=A_NBUF)
    def _main(kb):
        for b in range(A_NBUF):
            k = kb + b
            wait_in(col(k), b)

            @pl.when(k + LA < TCOL_MAIN)
            def _():
                fire_in(col(k + LA), (b + LA) % A_NBUF)

            rb = b % A_NRB

            @pl.when(k >= A_NRB)
            def _():
                wait_out(col(k - A_NRB), rb)

            transpose(b, rb)
            fire_out(col(k), rb)

    for k in range(TCOL_MAIN - A_NRB, TCOL_MAIN):
        wait_out(col(k), (k % A_NBUF) % A_NRB)

    @pl.when(wid < N_TCOL - TCOL_MAIN * NW)
    def _rem():
        k = TCOL_MAIN
        b = k % A_NBUF
        rb = b % A_NRB
        fire_in(col(k), b)
        wait_in(col(k), b)
        transpose(b, rb)
        fire_out(col(k), rb)
        wait_out(col(k), rb)


@jax.jit
def _reformat(tt):
    mesh = plsc.VectorSubcoreMesh(core_axis_name="c", subcore_axis_name="s")
    run = functools.partial(
        pl.kernel,
        out_type=jax.ShapeDtypeStruct((SCRATCH_ROWS * D_EMB,), jnp.float32),
        mesh=mesh,
        scratch_types=(
            [pltpu.VMEM((32, 128), jnp.float32)] * A_NBUF
            + [pltpu.VMEM((4096,), jnp.float32)] * A_NRB
            + [pltpu.SemaphoreType.DMA] * (A_NBUF + A_NRB)
        ),
        compiler_params=pltpu.CompilerParams(
            use_tc_tiling_on_sc=True, needs_layout_passes=False
        ),
    )(_reformat_kernel)
    return run(tt)


R_PER_W = N_I // NW
B_NBUF = 6
B_NT = 2
B_LA = 4


def _gather_kernel(scr_hbm, x4_hbm, o5_hbm, idx_v, rows_v, t_v, gsems, ssems):
    wid = lax.axis_index("s") * NC + lax.axis_index("c")

    pltpu.sync_copy(x4_hbm.at[:, wid], idx_v)

    def fire_g(j, b):
        jt, js = j // 8, j % 8
        pltpu.async_copy(
            scr_hbm.at[idx_v.at[jt, js]], rows_v.at[b], gsems.at[b]
        )

    def wait_g(j, b):
        jt, js = j // 8, j % 8
        pltpu.make_async_copy(
            scr_hbm.at[idx_v.at[jt, js]], rows_v.at[b], gsems.at[b]
        ).wait()

    def fire_s(j, tb):
        for dt in range(4):
            pltpu.async_copy(
                t_v.at[tb, pl.ds(dt * 1024, 1024)],
                o5_hbm.at[j, dt, wid],
                ssems.at[tb],
            )

    def wait_s(j, tb):
        for dt in range(4):
            pltpu.make_async_copy(
                t_v.at[tb, pl.ds(dt * 1024, 1024)],
                o5_hbm.at[j, dt, wid],
                ssems.at[tb],
            ).wait()

    l_idx = [_iota16() + l0 * 16 for l0 in range(8)]

    def transpose(b, tb):
        @plsc.parallel_loop(0, 32, unroll=4)
        def _t(d0):
            dmod = (d0 + _iota16()) & 31
            d128 = dmod * 128
            for l0 in range(8):
                x = plsc.load_gather(rows_v.at[b], [l_idx[l0], dmod])
                plsc.store_scatter(t_v.at[tb], [d128 + l_idx[l0]], x)

    for j in range(B_LA):
        fire_g(j, j % B_NBUF)

    @pl.loop(0, N_J - 2, step=B_NBUF)
    def _main(jb):
        for b in range(B_NBUF):
            j = jb + b
            wait_g(j, b)

            @pl.when(j + B_LA < N_J)
            def _():
                fire_g(j + B_LA, (b + B_LA) % B_NBUF)

            tb = b % B_NT

            @pl.when(j >= B_NT)
            def _():
                wait_s(j - B_NT, tb)

            transpose(b, tb)
            fire_s(j, tb)

    for j in range(N_J - 2, N_J):
        b = j % B_NBUF
        tb = b % B_NT
        wait_g(j, b)
        wait_s(j - B_NT, tb)
        transpose(b, tb)
        fire_s(j, tb)

    for j in range(N_J - B_NT, N_J):
        wait_s(j, (j % B_NBUF) % B_NT)


@jax.jit
def _emb(x, table):
    tt = table.T
    scratch = _reformat(tt)
    x4 = x.astype(jnp.int32).T.reshape(25, 8, 32, 128).transpose(0, 2, 1, 3)
    mesh = plsc.VectorSubcoreMesh(core_axis_name="c", subcore_axis_name="s")
    run = functools.partial(
        pl.kernel,
        out_type=jax.ShapeDtypeStruct((N_J, 4, NW, 1024), jnp.float32),
        mesh=mesh,
        scratch_types=[
            pltpu.VMEM((25, 8, 128), jnp.int32),
            pltpu.VMEM((B_NBUF, 128, 32), jnp.float32),
            pltpu.VMEM((B_NT, 4096), jnp.float32),
            pltpu.SemaphoreType.DMA((B_NBUF,)),
            pltpu.SemaphoreType.DMA((B_NT,)),
        ],
        compiler_params=pltpu.CompilerParams(
            use_tc_tiling_on_sc=False, needs_layout_passes=False
        ),
    )(_gather_kernel)
    o5 = run(scratch.reshape(SCRATCH_ROWS, D_EMB), x4)
    o6 = o5.reshape(N_J, 4, NW, 8, 128)
    return o6.transpose(2, 4, 0, 1, 3).reshape(N_I, N_J, D_EMB)


def kernel(x, table):
    return _emb(x, table)
